# Initial kernel scaffold; baseline (speedup 1.0000x reference)
import jax
import jax.numpy as jnp
from jax.experimental import pallas as pl


def kernel(x, y_points):
    raise NotImplementedError("write your pallas kernel here")


# speedup vs baseline: 9.5080x; 9.5080x over previous
import functools

import jax
import jax.numpy as jnp
from jax import lax
from jax.experimental import pallas as pl
from jax.experimental.pallas import tpu as pltpu
from jax.experimental.pallas import tpu_sc as plsc

N = 16384
SLOTS = 4
P = 33
DELTA = 1.0 / 4200.0 * 5.0
STEP = 2.0 * DELTA / (SLOTS - 1)

NC = 2
NS = 16
L = 16
NW = NC * NS
XC = N // NW
OC = XC * SLOTS
NVEC = OC // L
YPAD = 40


def _sc_body(x_hbm, y_hbm, out_hbm, x_v, y_v, out_v):
    wid = lax.axis_index("s") * NC + lax.axis_index("c")

    pltpu.sync_copy(x_hbm.at[pl.ds(wid * XC, XC)], x_v)
    pltpu.sync_copy(y_hbm, y_v)

    iota = lax.iota(jnp.int32, L)
    x_sel = lax.shift_right_logical(iota, 2)
    off_v = (iota & 3).astype(jnp.float32) * STEP - DELTA

    def body(j, _):
        ix = x_sel + j * (L // SLOTS)
        xv = plsc.load_gather(x_v, [ix])
        t = (xv + off_v) * 32.0
        k = t.astype(jnp.int32)
        k = jnp.minimum(jnp.maximum(k, 0), P - 2)
        y1 = plsc.load_gather(y_v, [k])
        y2 = plsc.load_gather(y_v, [k + 1])
        w = t - k.astype(jnp.float32)
        out_v[pl.ds(j * L, L)] = y1 + w * (y2 - y1)
        return _

    lax.fori_loop(0, NVEC, body, 0, unroll=4)

    pltpu.sync_copy(out_v, out_hbm.at[pl.ds(wid * OC, OC)])


@jax.jit
def kernel(x, y_points):
    y_flat = jnp.pad(y_points.reshape(-1), (0, YPAD - P))
    mesh = plsc.VectorSubcoreMesh(core_axis_name="c", subcore_axis_name="s")
    run = pl.kernel(
        _sc_body,
        out_type=jax.ShapeDtypeStruct((N * SLOTS,), jnp.float32),
        mesh=mesh,
        scratch_types=[
            pltpu.VMEM((XC,), jnp.float32),
            pltpu.VMEM((YPAD,), jnp.float32),
            pltpu.VMEM((OC,), jnp.float32),
        ],
        compiler_params=pltpu.CompilerParams(needs_layout_passes=False),
    )
    return run(x, y_flat).reshape(N, SLOTS)
